# baseline (speedup 1.0000x reference)
import jax
import jax.numpy as jnp
from jax.experimental import pallas as pl
from jax.experimental.pallas import tpu as pltpu

NUM_TOKENS = 16384
D_MODEL = 2048
NUM_EXPERTS = 16
TOP_K = 2

BT = 512


def _gate_block(x_ref, wt_ref, w_out_ref, idx_out_ref):
    logits = jnp.dot(x_ref[...], wt_ref[...], preferred_element_type=jnp.float32)
    iota = jax.lax.broadcasted_iota(jnp.int32, logits.shape, 1)
    l1 = jnp.max(logits, axis=1, keepdims=True)
    i1 = jnp.min(jnp.where(logits == l1, iota, NUM_EXPERTS), axis=1, keepdims=True)
    masked = jnp.where(iota == i1, -jnp.inf, logits)
    l2 = jnp.max(masked, axis=1, keepdims=True)
    i2 = jnp.min(jnp.where(masked == l2, iota, NUM_EXPERTS), axis=1, keepdims=True)

    e2 = jnp.exp(l2 - l1)
    s = 1.0 + e2
    w_out_ref[:, 0:1] = 1.0 / s
    w_out_ref[:, 1:2] = e2 / s
    idx_out_ref[:, 0:1] = i1
    idx_out_ref[:, 1:2] = i2


def kernel(x, W):
    wt = W.T
    grid = (NUM_TOKENS // BT,)
    w_out, idx_out = pl.pallas_call(
        _gate_block,
        grid=grid,
        in_specs=[
            pl.BlockSpec((BT, D_MODEL), lambda i: (i, 0)),
            pl.BlockSpec((D_MODEL, NUM_EXPERTS), lambda i: (0, 0)),
        ],
        out_specs=[
            pl.BlockSpec((BT, TOP_K), lambda i: (i, 0)),
            pl.BlockSpec((BT, TOP_K), lambda i: (i, 0)),
        ],
        out_shape=[
            jax.ShapeDtypeStruct((NUM_TOKENS, TOP_K), jnp.float32),
            jax.ShapeDtypeStruct((NUM_TOKENS, TOP_K), jnp.int32),
        ],
        compiler_params=pltpu.CompilerParams(
            dimension_semantics=("parallel",),
        ),
    )(x, wt)
    return (w_out, idx_out)


# speedup vs baseline: 1.2152x; 1.2152x over previous
import jax
import jax.numpy as jnp
from jax.experimental import pallas as pl
from jax.experimental.pallas import tpu as pltpu

NUM_TOKENS = 16384
D_MODEL = 2048
NUM_EXPERTS = 16
TOP_K = 2

BT = 2048


def _gate_block(x_ref, wt_ref, w_out_ref, idx_out_ref):
    logits = jnp.dot(x_ref[...], wt_ref[...], preferred_element_type=jnp.float32)
    iota = jax.lax.broadcasted_iota(jnp.int32, logits.shape, 1)
    l1 = jnp.max(logits, axis=1, keepdims=True)
    i1 = jnp.min(jnp.where(logits == l1, iota, NUM_EXPERTS), axis=1, keepdims=True)
    masked = jnp.where(iota == i1, -jnp.inf, logits)
    l2 = jnp.max(masked, axis=1, keepdims=True)
    i2 = jnp.min(jnp.where(masked == l2, iota, NUM_EXPERTS), axis=1, keepdims=True)

    e2 = jnp.exp(l2 - l1)
    s = 1.0 + e2
    w_out_ref[:, 0:1] = 1.0 / s
    w_out_ref[:, 1:2] = e2 / s
    idx_out_ref[:, 0:1] = i1
    idx_out_ref[:, 1:2] = i2


def kernel(x, W):
    wt = W.T
    grid = (NUM_TOKENS // BT,)
    w_out, idx_out = pl.pallas_call(
        _gate_block,
        grid=grid,
        in_specs=[
            pl.BlockSpec((BT, D_MODEL), lambda i: (i, 0)),
            pl.BlockSpec((D_MODEL, NUM_EXPERTS), lambda i: (0, 0)),
        ],
        out_specs=[
            pl.BlockSpec((BT, TOP_K), lambda i: (i, 0)),
            pl.BlockSpec((BT, TOP_K), lambda i: (i, 0)),
        ],
        out_shape=[
            jax.ShapeDtypeStruct((NUM_TOKENS, TOP_K), jnp.float32),
            jax.ShapeDtypeStruct((NUM_TOKENS, TOP_K), jnp.int32),
        ],
        compiler_params=pltpu.CompilerParams(
            dimension_semantics=("parallel",),
        ),
    )(x, wt)
    return (w_out, idx_out)
